# baseline (speedup 1.0000x reference)
import jax
import jax.numpy as jnp
from jax import lax
from jax.experimental import pallas as pl
from jax.experimental.pallas import tpu as pltpu
from jax.experimental.pallas import tpu_sc as plsc

H, W, D = 16384, 20, 64
S = 2 * W
OUT_W = W * D + S
TR = OUT_W // 8
RC = H // 128
NC, NS = 2, 16
NW = NC * NS
RPW = H // NW
RCW = RPW // 128


def _body(x_hbm, idxt_hbm, dat_hbm, o4_hbm, idx_v, g0, g1, b0, b1, sv,
          sg0, sg1, sb0, sb1):
  wid = lax.axis_index("s") * NC + lax.axis_index("c")
  wbase = wid * RPW
  rc0 = wid * RCW
  gbufs = (g0, g1)
  gsems = (sg0, sg1)
  bbufs = (b0, b1)
  bsems = (sb0, sb1)

  pltpu.sync_copy(idxt_hbm.at[:, pl.ds(wbase, RPW)], idx_v)

  def gather(j, b):
    pltpu.async_copy(x_hbm.at[idx_v.at[j]], gbufs[b], gsems[b])

  gather(0, 0)
  gather(1, 1)

  pltpu.sync_copy(dat_hbm.at[:, :, pl.ds(wbase, RPW)], sv)
  for jt in range(5):
    for rc_l in range(RCW):
      pltpu.sync_copy(sv.at[jt, :, pl.ds(128 * rc_l, 128)],
                      o4_hbm.at[160 + jt, rc0 + rc_l])

  lane = lax.iota(jnp.int32, 16)

  def o4dst(j, rc_l):
    return o4_hbm.at[pl.ds(pl.multiple_of(8 * j, 8), 8), rc0 + rc_l]

  def step(j, b):
    pltpu.make_async_copy(x_hbm.at[idx_v.at[j]], gbufs[b], gsems[b]).wait()
    for rc_l in range(RCW):
      bb = bbufs[rc_l % 2]
      sb = bsems[rc_l % 2]

      @pl.when(4 * j + rc_l >= 2)
      def _():
        pltpu.make_async_copy(bb, o4dst(j, rc_l), sb).wait()

      csv = lane & 7
      trv = [(lane >> 3) + 2 * q for q in range(4)]

      @plsc.parallel_loop(0, 128, 1, unroll=8)
      def _(rl):
        rlv = jnp.broadcast_to(rl, (16,))
        for q in range(4):
          v = gbufs[b][rc_l * 128 + rl, pl.ds(16 * q, 16)]
          plsc.store_scatter(bb, [trv[q], csv, rlv], v)
      pltpu.async_copy(bb, o4dst(j, rc_l), sb)

    @pl.when(j + 2 < W)
    def _():
      gather(j + 2, b)

  def pair(g, carry):
    step(2 * g, 0)
    step(2 * g + 1, 1)
    return carry

  lax.fori_loop(0, W // 2, pair, 0)
  pltpu.make_async_copy(b0, o4dst(W - 1, 2), sb0).wait()
  pltpu.make_async_copy(b1, o4dst(W - 1, 3), sb1).wait()


@jax.jit
def _run(x, idxt, dat):
  mesh = plsc.VectorSubcoreMesh(core_axis_name="c", subcore_axis_name="s")
  o4 = pl.kernel(
      _body,
      out_type=jax.ShapeDtypeStruct((TR, RC, 8, 128), jnp.float32),
      mesh=mesh,
      scratch_types=[
          pltpu.VMEM((W, RPW), jnp.int32),
          pltpu.VMEM((RPW, D), jnp.float32),
          pltpu.VMEM((RPW, D), jnp.float32),
          pltpu.VMEM((8, 8, 128), jnp.float32),
          pltpu.VMEM((8, 8, 128), jnp.float32),
          pltpu.VMEM((5, 8, RPW), jnp.float32),
          pltpu.SemaphoreType.DMA,
          pltpu.SemaphoreType.DMA,
          pltpu.SemaphoreType.DMA,
          pltpu.SemaphoreType.DMA,
      ],
      compiler_params=pltpu.CompilerParams(
          use_tc_tiling_on_sc=False, needs_layout_passes=False),
  )(x, idxt, dat)
  return o4.transpose(1, 3, 0, 2).reshape(H, OUT_W)


def kernel(x, idx, dis, angle_t):
  idxt = idx.astype(jnp.int32).T
  dat = jnp.concatenate([dis.T, angle_t.T], axis=0).reshape(5, 8, H)
  return _run(x, idxt, dat)


# speedup vs baseline: 1.1879x; 1.1879x over previous
import jax
import jax.numpy as jnp
from jax import lax
from jax.experimental import pallas as pl
from jax.experimental.pallas import tpu as pltpu
from jax.experimental.pallas import tpu_sc as plsc

H, W, D = 16384, 20, 64
S = 2 * W
OUT_W = W * D + S
NC, NS = 2, 16
NW = NC * NS
RPW = H // NW
NPAIR = W // 2


def _body(x_hbm, idxt_hbm, da_hbm, out_hbm, idx_v, rows0, rows1, da_v,
          sg0, sg1):
  wid = lax.axis_index("s") * NC + lax.axis_index("c")
  wbase = wid * RPW
  bufs = (rows0, rows1)
  sems = (sg0, sg1)
  pltpu.sync_copy(idxt_hbm.at[:, pl.ds(wbase, RPW)], idx_v)

  def gather(j, b):
    pltpu.async_copy(x_hbm.at[idx_v.at[j]], bufs[b], sems[b])

  gather(0, 0)
  gather(1, 1)
  pltpu.sync_copy(da_hbm.at[pl.ds(wbase, RPW), :], da_v)
  pltpu.sync_copy(da_v, out_hbm.at[pl.ds(wbase, RPW), pl.ds(W * D, S)])

  def pair(g, carry):
    for b in (0, 1):
      j = 2 * g + b
      pltpu.make_async_copy(x_hbm.at[idx_v.at[j]], bufs[b], sems[b]).wait()
      col = pl.multiple_of(j * D, D)
      pltpu.sync_copy(bufs[b],
                      out_hbm.at[pl.ds(wbase, RPW), pl.ds(col, D)])

      @pl.when(g < NPAIR - 1)
      def _():
        gather(j + 2, b)

    return carry

  lax.fori_loop(0, NPAIR, pair, 0)


@jax.jit
def _run(x, idxt, da):
  mesh = plsc.VectorSubcoreMesh(core_axis_name="c", subcore_axis_name="s")
  return pl.kernel(
      _body,
      out_type=jax.ShapeDtypeStruct((H, OUT_W), jnp.float32),
      mesh=mesh,
      scratch_types=[
          pltpu.VMEM((W, RPW), jnp.int32),
          pltpu.VMEM((RPW, D), jnp.float32),
          pltpu.VMEM((RPW, D), jnp.float32),
          pltpu.VMEM((RPW, S), jnp.float32),
          pltpu.SemaphoreType.DMA,
          pltpu.SemaphoreType.DMA,
      ],
      compiler_params=pltpu.CompilerParams(use_tc_tiling_on_sc=False),
  )(x, idxt, da)


def kernel(x, idx, dis, angle_t):
  idxt = idx.astype(jnp.int32).T
  da = jnp.concatenate([dis, angle_t], axis=1)
  return _run(x, idxt, da)
